# baseline (speedup 1.0000x reference)
import functools

import numpy as np
import jax
import jax.numpy as jnp
from jax import lax
from jax.experimental import pallas as pl
from jax.experimental.pallas import tpu as pltpu
from jax.experimental.pallas import tpu_sc as plsc

_B = 1024
_N = 4096


def _make_perm() -> np.ndarray:
    np.random.seed(42)
    ind = np.arange(_N)
    np.random.shuffle(ind)
    return ind.astype(np.int32)


_PERM = _make_perm()

_NC = 2
_NS = 16
_L = 16
_NW = _NC * _NS
_ROWS_PER_W = _B // _NW
_C = 8
_NCHUNK = _ROWS_PER_W // _C


@functools.cache
def _build():
    mesh = plsc.VectorSubcoreMesh(core_axis_name="c", subcore_axis_name="s")

    @functools.partial(
        pl.kernel,
        mesh=mesh,
        out_type=jax.ShapeDtypeStruct((_B * _N,), jnp.float32),
        scratch_types=[
            pltpu.VMEM((_N,), jnp.int32),
            pltpu.VMEM((_C * _N,), jnp.float32),
            pltpu.VMEM((_C * _N,), jnp.float32),
        ],
        compiler_params=pltpu.CompilerParams(needs_layout_passes=False),
    )
    def shuffle_sc(x_hbm, perm_hbm, out_hbm, idx_v, in_v, out_v):
        wid = lax.axis_index("s") * _NC + lax.axis_index("c")
        pltpu.sync_copy(perm_hbm, idx_v)
        base0 = wid * _ROWS_PER_W * _N

        def chunk_body(c, carry):
            base = base0 + c * (_C * _N)
            pltpu.sync_copy(x_hbm.at[pl.ds(base, _C * _N)], in_v)

            def jb_body(jb, carry2):
                col = jb * _L
                idx = idx_v[pl.ds(col, _L)]
                for r in range(_C):
                    out_v[pl.ds(r * _N + col, _L)] = plsc.load_gather(
                        in_v, [idx + (r * _N)])
                return carry2

            lax.fori_loop(0, _N // _L, jb_body, 0)
            pltpu.sync_copy(out_v, out_hbm.at[pl.ds(base, _C * _N)])
            return carry

        lax.fori_loop(0, _NCHUNK, chunk_body, 0)

    return shuffle_sc


def kernel(x):
    return _build()(x.reshape(-1), _PERM).reshape(_B, _N)


# speedup vs baseline: 1.0862x; 1.0862x over previous
import functools

import numpy as np
import jax
import jax.numpy as jnp
from jax import lax
from jax.experimental import pallas as pl
from jax.experimental.pallas import tpu as pltpu
from jax.experimental.pallas import tpu_sc as plsc

_B = 1024
_N = 4096


def _make_perm() -> np.ndarray:
    np.random.seed(42)
    ind = np.arange(_N)
    np.random.shuffle(ind)
    return ind.astype(np.int32)


_PERM = _make_perm()

_NC = 2
_NS = 16
_L = 16
_NW = _NC * _NS
_ROWS_PER_W = _B // _NW
_C = 4
_CN = _C * _N
_NCHUNK = _ROWS_PER_W // _C
_UNROLL = 4


@functools.cache
def _build():
    mesh = plsc.VectorSubcoreMesh(core_axis_name="c", subcore_axis_name="s")

    @functools.partial(
        pl.kernel,
        mesh=mesh,
        out_type=jax.ShapeDtypeStruct((_B * _N,), jnp.float32),
        scratch_types=[
            pltpu.VMEM((_N,), jnp.int32),
            pltpu.VMEM((_CN,), jnp.float32),
            pltpu.VMEM((_CN,), jnp.float32),
            pltpu.VMEM((_CN,), jnp.float32),
            pltpu.VMEM((_CN,), jnp.float32),
            pltpu.SemaphoreType.DMA,
            pltpu.SemaphoreType.DMA,
            pltpu.SemaphoreType.DMA,
            pltpu.SemaphoreType.DMA,
        ],
        compiler_params=pltpu.CompilerParams(needs_layout_passes=False),
    )
    def shuffle_sc(x_hbm, perm_hbm, out_hbm, idx_v, in0, in1, o0, o1,
                   si0, si1, so0, so1):
        wid = lax.axis_index("s") * _NC + lax.axis_index("c")
        pltpu.sync_copy(perm_hbm, idx_v)
        base0 = wid * (_ROWS_PER_W * _N)

        in_bufs, out_bufs = [in0, in1], [o0, o1]
        in_sems, out_sems = [si0, si1], [so0, so1]

        def start_in(g, b):
            return pltpu.async_copy(
                x_hbm.at[pl.ds(base0 + g * _CN, _CN)], in_bufs[b], in_sems[b])

        h_in = [start_in(0, 0), start_in(1, 1)]
        h_out = [None, None]

        for g in range(_NCHUNK):
            b = g & 1
            h_in[b].wait()
            if h_out[b] is not None:
                h_out[b].wait()
            src, dst = in_bufs[b], out_bufs[b]

            @plsc.parallel_loop(0, _N // _L, unroll=_UNROLL)
            def _gather(jb):
                col = jb * _L
                idx = idx_v[pl.ds(col, _L)]
                for r in range(_C):
                    dst[pl.ds(r * _N + col, _L)] = plsc.load_gather(
                        src, [idx + (r * _N)])

            if g + 2 < _NCHUNK:
                h_in[b] = start_in(g + 2, b)
            h_out[b] = pltpu.async_copy(
                dst, out_hbm.at[pl.ds(base0 + g * _CN, _CN)], out_sems[b])

        h_out[0].wait()
        h_out[1].wait()

    return shuffle_sc


def kernel(x):
    return _build()(x.reshape(-1), _PERM).reshape(_B, _N)


# speedup vs baseline: 2.0914x; 1.9255x over previous
import functools

import numpy as np
import jax
import jax.numpy as jnp
from jax import lax
from jax.experimental import pallas as pl
from jax.experimental.pallas import tpu as pltpu
from jax.experimental.pallas import tpu_sc as plsc

_B = 1024
_N = 4096


def _make_perm() -> np.ndarray:
    np.random.seed(42)
    ind = np.arange(_N)
    np.random.shuffle(ind)
    return ind.astype(np.int32)


_PERM = _make_perm()

_NC = 2
_NS = 16
_L = 16
_NW = _NC * _NS
_ROWS_PER_W = _B // _NW
_C = 4
_NCHUNK = _ROWS_PER_W // _C
_UNROLL = 4


@functools.cache
def _build():
    mesh = plsc.VectorSubcoreMesh(core_axis_name="c", subcore_axis_name="s")

    @functools.partial(
        pl.kernel,
        mesh=mesh,
        out_type=jax.ShapeDtypeStruct((_B, _N), jnp.float32),
        scratch_types=[
            pltpu.VMEM((_N,), jnp.int32),
            pltpu.VMEM((_C, _N), jnp.float32),
            pltpu.VMEM((_C, _N), jnp.float32),
            pltpu.VMEM((_C, _N), jnp.float32),
            pltpu.VMEM((_C, _N), jnp.float32),
            pltpu.SemaphoreType.DMA,
            pltpu.SemaphoreType.DMA,
            pltpu.SemaphoreType.DMA,
            pltpu.SemaphoreType.DMA,
        ],
        compiler_params=pltpu.CompilerParams(needs_layout_passes=False),
    )
    def shuffle_sc(x_hbm, perm_hbm, out_hbm, idx_v, in0, in1, o0, o1,
                   si0, si1, so0, so1):
        wid = lax.axis_index("s") * _NC + lax.axis_index("c")
        pltpu.sync_copy(perm_hbm, idx_v)
        row0 = wid * _ROWS_PER_W

        in_bufs, out_bufs = [in0, in1], [o0, o1]
        in_sems, out_sems = [si0, si1], [so0, so1]

        def start_in(g, b):
            return pltpu.async_copy(
                x_hbm.at[pl.ds(row0 + g * _C, _C)], in_bufs[b], in_sems[b])

        h_in = [start_in(0, 0), start_in(1, 1)]
        h_out = [None, None]

        for g in range(_NCHUNK):
            b = g & 1
            h_in[b].wait()
            if h_out[b] is not None:
                h_out[b].wait()
            src, dst = in_bufs[b], out_bufs[b]

            @plsc.parallel_loop(0, _N // _L, unroll=_UNROLL)
            def _gather(jb):
                col = jb * _L
                idx = idx_v[pl.ds(col, _L)]
                for r in range(_C):
                    row_idx = jnp.full((_L,), r, jnp.int32)
                    dst[r, pl.ds(col, _L)] = plsc.load_gather(
                        src, [row_idx, idx])

            if g + 2 < _NCHUNK:
                h_in[b] = start_in(g + 2, b)
            h_out[b] = pltpu.async_copy(
                dst, out_hbm.at[pl.ds(row0 + g * _C, _C)], out_sems[b])

        h_out[0].wait()
        h_out[1].wait()

    return shuffle_sc


def kernel(x):
    return _build()(x, _PERM)


# speedup vs baseline: 2.1764x; 1.0406x over previous
import functools

import numpy as np
import jax
import jax.numpy as jnp
from jax import lax
from jax.experimental import pallas as pl
from jax.experimental.pallas import tpu as pltpu
from jax.experimental.pallas import tpu_sc as plsc

_B = 1024
_N = 4096


def _make_perm() -> np.ndarray:
    np.random.seed(42)
    ind = np.arange(_N)
    np.random.shuffle(ind)
    return ind.astype(np.int32)


_PERM = _make_perm()

_NC = 2
_NS = 16
_L = 16
_NW = _NC * _NS
_ROWS_PER_W = _B // _NW
_C = 4
_NCHUNK = _ROWS_PER_W // _C
_UNROLL = 4


@functools.cache
def _build():
    mesh = plsc.VectorSubcoreMesh(core_axis_name="c", subcore_axis_name="s")

    @functools.partial(
        pl.kernel,
        mesh=mesh,
        out_type=jax.ShapeDtypeStruct((_B, _N), jnp.float32),
        scratch_types=[
            pltpu.VMEM((_N,), jnp.int32),
            pltpu.VMEM((2, _C, _N), jnp.float32),
            pltpu.VMEM((2, _C, _N), jnp.float32),
            pltpu.SemaphoreType.DMA,
            pltpu.SemaphoreType.DMA,
            pltpu.SemaphoreType.DMA,
            pltpu.SemaphoreType.DMA,
        ],
        compiler_params=pltpu.CompilerParams(needs_layout_passes=False),
    )
    def shuffle_sc(x_hbm, perm_hbm, out_hbm, idx_v, in_v, out_v,
                   si0, si1, so0, so1):
        wid = lax.axis_index("s") * _NC + lax.axis_index("c")
        pltpu.sync_copy(perm_hbm, idx_v)
        row0 = wid * _ROWS_PER_W

        def in_copy(g, slot, sem):
            return pltpu.make_async_copy(
                x_hbm.at[pl.ds(row0 + g * _C, _C)], in_v.at[slot], sem)

        def out_copy(g, slot, sem):
            return pltpu.make_async_copy(
                out_v.at[slot], out_hbm.at[pl.ds(row0 + g * _C, _C)], sem)

        in_copy(0, 0, si0).start()
        in_copy(1, 1, si1).start()

        def chunk_body(g, carry):
            b = g & 1

            @pl.when(b == 0)
            def _():
                in_copy(g, 0, si0).wait()

            @pl.when(b == 1)
            def _():
                in_copy(g, 1, si1).wait()

            @pl.when((g >= 2) & (b == 0))
            def _():
                out_copy(g - 2, 0, so0).wait()

            @pl.when((g >= 2) & (b == 1))
            def _():
                out_copy(g - 2, 1, so1).wait()

            b_idx = jnp.full((_L,), b, jnp.int32)

            @plsc.parallel_loop(0, _N // _L, unroll=_UNROLL)
            def _gather(jb):
                col = jb * _L
                idx = idx_v[pl.ds(col, _L)]
                for r in range(_C):
                    row_idx = jnp.full((_L,), r, jnp.int32)
                    out_v[b, r, pl.ds(col, _L)] = plsc.load_gather(
                        in_v, [b_idx, row_idx, idx])

            @pl.when((g + 2 < _NCHUNK) & (b == 0))
            def _():
                in_copy(g + 2, 0, si0).start()

            @pl.when((g + 2 < _NCHUNK) & (b == 1))
            def _():
                in_copy(g + 2, 1, si1).start()

            @pl.when(b == 0)
            def _():
                out_copy(g, 0, so0).start()

            @pl.when(b == 1)
            def _():
                out_copy(g, 1, so1).start()

            return carry

        lax.fori_loop(0, _NCHUNK, chunk_body, 0)
        out_copy(_NCHUNK - 2, 0, so0).wait()
        out_copy(_NCHUNK - 1, 1, so1).wait()

    return shuffle_sc


def kernel(x):
    return _build()(x, _PERM)


# speedup vs baseline: 2.2593x; 1.0381x over previous
import functools

import numpy as np
import jax
import jax.numpy as jnp
from jax import lax
from jax.experimental import pallas as pl
from jax.experimental.pallas import tpu as pltpu
from jax.experimental.pallas import tpu_sc as plsc

_B = 1024
_N = 4096


def _make_perm() -> np.ndarray:
    np.random.seed(42)
    ind = np.arange(_N)
    np.random.shuffle(ind)
    return ind.astype(np.int32)


_PERM = _make_perm()

_NC = 2
_NS = 16
_L = 16
_NW = _NC * _NS
_ROWS_PER_W = _B // _NW
_C = 4
_NCHUNK = _ROWS_PER_W // _C
_SLOTS = 3
_UNROLL = 4


@functools.cache
def _build():
    mesh = plsc.VectorSubcoreMesh(core_axis_name="c", subcore_axis_name="s")

    @functools.partial(
        pl.kernel,
        mesh=mesh,
        out_type=jax.ShapeDtypeStruct((_B, _N), jnp.float32),
        scratch_types=[
            pltpu.VMEM((_N,), jnp.int32),
            pltpu.VMEM((_SLOTS, _C, _N), jnp.float32),
            pltpu.VMEM((_SLOTS, _C, _N), jnp.float32),
            [pltpu.SemaphoreType.DMA] * _SLOTS,
            [pltpu.SemaphoreType.DMA] * _SLOTS,
        ],
        compiler_params=pltpu.CompilerParams(needs_layout_passes=False),
    )
    def shuffle_sc(x_hbm, perm_hbm, out_hbm, idx_v, in_v, out_v,
                   in_sems, out_sems):
        wid = lax.axis_index("s") * _NC + lax.axis_index("c")
        row0 = wid * _ROWS_PER_W

        def in_copy(g, slot):
            return pltpu.make_async_copy(
                x_hbm.at[pl.ds(row0 + g * _C, _C)], in_v.at[slot],
                in_sems[slot])

        def out_copy(g, slot):
            return pltpu.make_async_copy(
                out_v.at[slot], out_hbm.at[pl.ds(row0 + g * _C, _C)],
                out_sems[slot])

        for s in range(_SLOTS):
            in_copy(s, s).start()
        pltpu.sync_copy(perm_hbm, idx_v)

        def chunk_body(g, carry):
            b = g % _SLOTS

            for s in range(_SLOTS):
                @pl.when(b == s)
                def _(s=s):
                    in_copy(g, s).wait()

                @pl.when((g >= _SLOTS) & (b == s))
                def _(s=s):
                    out_copy(g - _SLOTS, s).wait()

            b_idx = jnp.full((_L,), b, jnp.int32)

            @plsc.parallel_loop(0, _N // _L, unroll=_UNROLL)
            def _gather(jb):
                col = jb * _L
                idx = idx_v[pl.ds(col, _L)]
                for r in range(_C):
                    row_idx = jnp.full((_L,), r, jnp.int32)
                    out_v[b, r, pl.ds(col, _L)] = plsc.load_gather(
                        in_v, [b_idx, row_idx, idx])

            for s in range(_SLOTS):
                @pl.when(b == s)
                def _(s=s):
                    out_copy(g, s).start()

                @pl.when((g + _SLOTS < _NCHUNK) & (b == s))
                def _(s=s):
                    in_copy(g + _SLOTS, s).start()

            return carry

        lax.fori_loop(0, _NCHUNK, chunk_body, 0)
        for g in range(_NCHUNK - _SLOTS, _NCHUNK):
            out_copy(g, g % _SLOTS).wait()

    return shuffle_sc


def kernel(x):
    return _build()(x, _PERM)


# speedup vs baseline: 2.2690x; 1.0043x over previous
import functools

import numpy as np
import jax
import jax.numpy as jnp
from jax import lax
from jax.experimental import pallas as pl
from jax.experimental.pallas import tpu as pltpu
from jax.experimental.pallas import tpu_sc as plsc

_B = 1024
_N = 4096


def _make_perm() -> np.ndarray:
    np.random.seed(42)
    ind = np.arange(_N)
    np.random.shuffle(ind)
    return ind.astype(np.int32)


_PERM = _make_perm()

_NC = 2
_NS = 16
_L = 16
_NW = _NC * _NS
_ROWS_PER_W = _B // _NW
_C = 4
_NCHUNK = _ROWS_PER_W // _C
_SLOTS = 3
_UNROLL = 2


@functools.cache
def _build():
    mesh = plsc.VectorSubcoreMesh(core_axis_name="c", subcore_axis_name="s")

    @functools.partial(
        pl.kernel,
        mesh=mesh,
        out_type=jax.ShapeDtypeStruct((_B, _N), jnp.float32),
        scratch_types=[
            pltpu.VMEM((_N,), jnp.int32),
            pltpu.VMEM((_SLOTS, _C, _N), jnp.float32),
            pltpu.VMEM((_SLOTS, _C, _N), jnp.float32),
            pltpu.SemaphoreType.DMA((_SLOTS,)),
            pltpu.SemaphoreType.DMA((_SLOTS,)),
        ],
        compiler_params=pltpu.CompilerParams(needs_layout_passes=False),
    )
    def shuffle_sc(x_hbm, perm_hbm, out_hbm, idx_v, in_v, out_v,
                   in_sems, out_sems):
        wid = lax.axis_index("s") * _NC + lax.axis_index("c")
        row0 = wid * _ROWS_PER_W

        def in_copy(g, slot):
            return pltpu.make_async_copy(
                x_hbm.at[pl.ds(row0 + g * _C, _C)], in_v.at[slot],
                in_sems.at[slot])

        def out_copy(g, slot):
            return pltpu.make_async_copy(
                out_v.at[slot], out_hbm.at[pl.ds(row0 + g * _C, _C)],
                out_sems.at[slot])

        for s in range(_SLOTS):
            in_copy(s, s).start()
        pltpu.sync_copy(perm_hbm, idx_v)

        def chunk_body(g, carry):
            b = g % _SLOTS
            in_copy(g, b).wait()

            @pl.when(g >= _SLOTS)
            def _():
                out_copy(g - _SLOTS, b).wait()

            b_idx = jnp.full((_L,), b, jnp.int32)

            @plsc.parallel_loop(0, _N // _L, unroll=_UNROLL)
            def _gather(jb):
                col = jb * _L
                idx = idx_v[pl.ds(col, _L)]
                for r in range(_C):
                    row_idx = jnp.full((_L,), r, jnp.int32)
                    out_v[b, r, pl.ds(col, _L)] = plsc.load_gather(
                        in_v, [b_idx, row_idx, idx])

            out_copy(g, b).start()

            @pl.when(g + _SLOTS < _NCHUNK)
            def _():
                in_copy(g + _SLOTS, b).start()

            return carry

        lax.fori_loop(0, _NCHUNK, chunk_body, 0)
        for g in range(_NCHUNK - _SLOTS, _NCHUNK):
            out_copy(g, g % _SLOTS).wait()

    return shuffle_sc


def kernel(x):
    return _build()(x, _PERM)
